# baseline (speedup 1.0000x reference)
import functools

import jax
import jax.numpy as jnp
from jax import lax
from jax.experimental import pallas as pl
from jax.experimental.pallas import tpu as pltpu
from jax.experimental.pallas import tpu_sc as plsc

K = 16
CHUNK = 256
RL = 128


def _topk_body(pc_blk_ref, pc_all_ref, idx_ref, dist_ref):
    b = pl.program_id(0)
    n = pc_all_ref.shape[1]
    r_total = pc_blk_ref.shape[1]
    nsb = r_total // RL
    nch = n // CHUNK

    for sb in range(nsb):
        a = pc_blk_ref[0, pl.ds(sb * RL, RL), :]

        for c in range(nch):
            t_c = pc_all_ref[0, pl.ds(c * CHUNK, CHUNK), :]
            inner = lax.dot_general(
                t_c, a, (((1,), (1,)), ((), ())),
                preferred_element_type=jnp.float32,
            )
            sq = jnp.sum(t_c * t_c, axis=1, keepdims=True)
            dist_ref[sb, c] = inner + inner - sq

        sub = lax.broadcasted_iota(jnp.int32, (CHUNK, RL), 0).astype(jnp.float32)
        nf = jnp.float32(n)
        m_prev = jnp.full((1, RL), jnp.inf, jnp.float32)
        am_prev = jnp.full((1, RL), -1.0, jnp.float32)
        for k in range(K):
            if k > 0:
                mu = lax.bitcast_convert_type(m_prev, jnp.int32)
                m_up = lax.bitcast_convert_type(
                    mu + jnp.where(mu >= 0, 1, -1), jnp.float32
                )
            m_run = jnp.full((1, RL), -jnp.inf, jnp.float32)
            am_run = jnp.full((1, RL), nf, jnp.float32)
            for c in range(nch):
                x = dist_ref[sb, c]
                if k == 0:
                    xm = x
                else:
                    am_loc = am_prev - jnp.float32(c * CHUNK)
                    thr = jnp.where(sub > am_loc, m_up, m_prev)
                    xm = jnp.where(x < thr, x, -jnp.inf)
                cmax = jnp.max(xm, axis=0, keepdims=True)
                cloc = jnp.min(
                    jnp.where(xm == cmax, sub, nf), axis=0, keepdims=True
                )
                cidx = cloc + jnp.float32(c * CHUNK)
                better = (cmax > m_run) | ((cmax == m_run) & (cidx < am_run))
                m_run = jnp.where(better, cmax, m_run)
                am_run = jnp.where(better, cidx, am_run)
            m_prev, am_prev = m_run, am_run
            idx_ref[0, k, pl.ds(sb * RL, RL)] = (
                am_prev[0].astype(jnp.int32) + b * n)


def _topk_indices(pc, block_r):
    b, n, d = pc.shape
    grid = (b, n // block_r)
    return pl.pallas_call(
        _topk_body,
        grid=grid,
        in_specs=[
            pl.BlockSpec((1, block_r, d), lambda i, j: (i, j, 0)),
            pl.BlockSpec((1, n, d), lambda i, j: (i, 0, 0)),
        ],
        out_specs=pl.BlockSpec((1, K, block_r), lambda i, j: (i, 0, j)),
        out_shape=jax.ShapeDtypeStruct((b, K, n), jnp.int32),
        scratch_shapes=[
            pltpu.VMEM((block_r // RL, n // CHUNK, CHUNK, RL), jnp.float32)],
    )(pc, pc)


def _sc_gather(pc_flat, nn_flat, rows_c):
    nrow, d = pc_flat.shape
    total = nn_flat.shape[0]
    num_cores, num_subcores = 2, 16
    nw = num_cores * num_subcores
    rows_per_w = nrow // nw
    n_chunks = rows_per_w // rows_c
    g = rows_c * K
    ng = g // 128
    nn2 = nn_flat.reshape(total // 128, 128)
    mesh = plsc.VectorSubcoreMesh(
        core_axis_name="c", subcore_axis_name="s",
        num_cores=num_cores, num_subcores=num_subcores,
    )

    @functools.partial(
        pl.kernel,
        out_type=jax.ShapeDtypeStruct((total, 2 * d), jnp.float32),
        mesh=mesh,
        scratch_types=[
            pltpu.VMEM((ng, 128), jnp.int32), pltpu.VMEM((ng, 128), jnp.int32),
            pltpu.VMEM((g, d), jnp.float32), pltpu.VMEM((g, d), jnp.float32),
            pltpu.VMEM((rows_c, d), jnp.float32),
            pltpu.VMEM((rows_c, d), jnp.float32),
            pltpu.VMEM((g, 2 * d), jnp.float32),
            pltpu.VMEM((g, 2 * d), jnp.float32),
            pltpu.SemaphoreType.DMA, pltpu.SemaphoreType.DMA,
            pltpu.SemaphoreType.DMA, pltpu.SemaphoreType.DMA,
            pltpu.SemaphoreType.DMA, pltpu.SemaphoreType.DMA,
            pltpu.SemaphoreType.DMA, pltpu.SemaphoreType.DMA,
        ],
        compiler_params=pltpu.CompilerParams(use_tc_tiling_on_sc=False),
    )
    def sc_kernel(pc_hbm, nn2_hbm, out_hbm,
                  idx0, idx1, nbr0, nbr1, cen0, cen1, out0, out1,
                  sg0, sg1, so0, so1, si0, si1, sc0, sc1):
        wid = lax.axis_index("s") * num_cores + lax.axis_index("c")
        row_base = wid * rows_per_w
        idx_v, nbr_v, cen_v, out_v = (idx0, idx1), (nbr0, nbr1), (cen0, cen1), (out0, out1)
        sg, so, si, sc = (sg0, sg1), (so0, so1), (si0, si1), (sc0, sc1)

        def issue_idx(ci, bf):
            r0 = row_base + ci * rows_c
            pltpu.async_copy(nn2_hbm.at[pl.ds(r0 * K // 128, ng)], idx_v[bf], si[bf])

        def wait_idx(bf):
            pltpu.make_async_copy(nn2_hbm.at[pl.ds(0, ng)], idx_v[bf], si[bf]).wait()

        def issue_gather_cen(ci, bf):
            r0 = row_base + ci * rows_c
            for q2 in range(ng):
                pltpu.async_copy(
                    pc_hbm.at[idx_v[bf].at[q2]],
                    nbr_v[bf].at[pl.ds(q2 * 128, 128)], sg[bf])
            pltpu.async_copy(pc_hbm.at[pl.ds(r0, rows_c)], cen_v[bf], sc[bf])

        def wait_gather(bf):
            for q2 in range(ng):
                pltpu.make_async_copy(
                    pc_hbm.at[idx_v[bf].at[q2]],
                    nbr_v[bf].at[pl.ds(q2 * 128, 128)], sg[bf]).wait()

        def wait_cen(bf):
            pltpu.make_async_copy(
                pc_hbm.at[pl.ds(0, rows_c)], cen_v[bf], sc[bf]).wait()

        issue_idx(0, 0)
        issue_idx(1, 1)
        wait_idx(0)
        issue_gather_cen(0, 0)

        def outer(h, _):
            for bf in range(2):
                i = h * 2 + bf
                r0 = row_base + i * rows_c
                wait_gather(bf)
                issue_idx(jnp.minimum(i + 2, n_chunks - 1), bf)
                wait_idx(bf ^ 1)
                issue_gather_cen(jnp.minimum(i + 1, n_chunks - 1), bf ^ 1)

                @pl.when(h >= 1)
                def _():
                    pltpu.make_async_copy(
                        out_v[bf], out_hbm.at[pl.ds(r0 * K, g)], so[bf]).wait()

                wait_cen(bf)

                def edge(rr, _):
                    base = rr * K
                    cens = [cen_v[bf][rr, pl.ds(q * 16, 16)]
                            for q in range(d // 16)]
                    for kk in range(K):
                        j = base + kk
                        for q in range(d // 16):
                            nbr = nbr_v[bf][j, pl.ds(q * 16, 16)]
                            out_v[bf][j, pl.ds(q * 16, 16)] = cens[q]
                            out_v[bf][j, pl.ds(d + q * 16, 16)] = nbr - cens[q]
                    return 0

                lax.fori_loop(0, rows_c, edge, 0)
                pltpu.async_copy(out_v[bf], out_hbm.at[pl.ds(r0 * K, g)], so[bf])
            return 0

        lax.fori_loop(0, n_chunks // 2, outer, 0)
        last0 = row_base + (n_chunks - 2) * rows_c
        last1 = row_base + (n_chunks - 1) * rows_c
        pltpu.make_async_copy(out_v[0], out_hbm.at[pl.ds(last0 * K, g)], so[0]).wait()
        pltpu.make_async_copy(out_v[1], out_hbm.at[pl.ds(last1 * K, g)], so[1]).wait()
        wait_gather(0)
        wait_cen(0)
        wait_idx(1)

    return sc_kernel(pc_flat, nn2)


def kernel(inputs):
    known_axes = tuple(i for i, s in enumerate(inputs.shape) if s == 1)
    pc = jnp.squeeze(inputs, axis=known_axes) if known_axes else inputs
    b, n, d = pc.shape

    halves = []
    for h in range(2):
        pch = pc[h * (b // 2):(h + 1) * (b // 2)]
        nn_t = _topk_indices(pch, block_r=512)
        nn = jnp.transpose(nn_t, (0, 2, 1)).reshape(b // 2 * n * K)
        halves.append(_sc_gather(pch.reshape(b // 2 * n, d), nn, rows_c=16))
    out = jnp.concatenate(halves, axis=0)
    return out.reshape(b, n, K, 2 * d)


# speedup vs baseline: 1.0011x; 1.0011x over previous
import functools

import jax
import jax.numpy as jnp
from jax import lax
from jax.experimental import pallas as pl
from jax.experimental.pallas import tpu as pltpu
from jax.experimental.pallas import tpu_sc as plsc

K = 16
CHUNK = 256
RL = 128


def _topk_body(pc_blk_ref, pc_all_ref, idx_ref, dist_ref):
    b = pl.program_id(0)
    n = pc_all_ref.shape[1]
    r_total = pc_blk_ref.shape[1]
    nsb = r_total // RL
    nch = n // CHUNK

    for sb in range(nsb):
        a = pc_blk_ref[0, pl.ds(sb * RL, RL), :]

        for c in range(nch):
            t_c = pc_all_ref[0, pl.ds(c * CHUNK, CHUNK), :]
            inner = lax.dot_general(
                t_c, a, (((1,), (1,)), ((), ())),
                preferred_element_type=jnp.float32,
            )
            sq = jnp.sum(t_c * t_c, axis=1, keepdims=True)
            dist_ref[sb, c] = inner + inner - sq

        sub = lax.broadcasted_iota(jnp.int32, (CHUNK, RL), 0).astype(jnp.float32)
        nf = jnp.float32(n)
        m_prev = jnp.full((1, RL), jnp.inf, jnp.float32)
        am_prev = jnp.full((1, RL), -1.0, jnp.float32)
        for k in range(K):
            if k > 0:
                mu = lax.bitcast_convert_type(m_prev, jnp.int32)
                m_up = lax.bitcast_convert_type(
                    mu + jnp.where(mu >= 0, 1, -1), jnp.float32
                )
            m_run = jnp.full((1, RL), -jnp.inf, jnp.float32)
            am_run = jnp.full((1, RL), nf, jnp.float32)
            for c in range(nch):
                x = dist_ref[sb, c]
                if k == 0:
                    xm = x
                else:
                    am_loc = am_prev - jnp.float32(c * CHUNK)
                    thr = jnp.where(sub > am_loc, m_up, m_prev)
                    xm = jnp.where(x < thr, x, -jnp.inf)
                cmax = jnp.max(xm, axis=0, keepdims=True)
                cloc = jnp.min(
                    jnp.where(xm == cmax, sub, nf), axis=0, keepdims=True
                )
                cidx = cloc + jnp.float32(c * CHUNK)
                better = (cmax > m_run) | ((cmax == m_run) & (cidx < am_run))
                m_run = jnp.where(better, cmax, m_run)
                am_run = jnp.where(better, cidx, am_run)
            m_prev, am_prev = m_run, am_run
            idx_ref[0, k, pl.ds(sb * RL, RL)] = (
                am_prev[0].astype(jnp.int32) + b * n)


def _topk_indices(pc, block_r):
    b, n, d = pc.shape
    grid = (b, n // block_r)
    return pl.pallas_call(
        _topk_body,
        grid=grid,
        in_specs=[
            pl.BlockSpec((1, block_r, d), lambda i, j: (i, j, 0)),
            pl.BlockSpec((1, n, d), lambda i, j: (i, 0, 0)),
        ],
        out_specs=pl.BlockSpec((1, K, block_r), lambda i, j: (i, 0, j)),
        out_shape=jax.ShapeDtypeStruct((b, K, n), jnp.int32),
        scratch_shapes=[
            pltpu.VMEM((block_r // RL, n // CHUNK, CHUNK, RL), jnp.float32)],
    )(pc, pc)


def _sc_gather(pc_flat, nn_flat, rows_c):
    nrow, d = pc_flat.shape
    total = nn_flat.shape[0]
    num_cores, num_subcores = 2, 16
    nw = num_cores * num_subcores
    rows_per_w = nrow // nw
    n_chunks = rows_per_w // rows_c
    g = rows_c * K
    ng = g // 128
    nn2 = nn_flat.reshape(total // 128, 128)
    mesh = plsc.VectorSubcoreMesh(
        core_axis_name="c", subcore_axis_name="s",
        num_cores=num_cores, num_subcores=num_subcores,
    )

    @functools.partial(
        pl.kernel,
        out_type=jax.ShapeDtypeStruct((total, 2 * d), jnp.float32),
        mesh=mesh,
        scratch_types=[
            pltpu.VMEM((ng, 128), jnp.int32), pltpu.VMEM((ng, 128), jnp.int32),
            pltpu.VMEM((g, d), jnp.float32), pltpu.VMEM((g, d), jnp.float32),
            pltpu.VMEM((rows_c, d), jnp.float32),
            pltpu.VMEM((rows_c, d), jnp.float32),
            pltpu.VMEM((g, 2 * d), jnp.float32),
            pltpu.VMEM((g, 2 * d), jnp.float32),
            pltpu.SemaphoreType.DMA, pltpu.SemaphoreType.DMA,
            pltpu.SemaphoreType.DMA, pltpu.SemaphoreType.DMA,
            pltpu.SemaphoreType.DMA, pltpu.SemaphoreType.DMA,
            pltpu.SemaphoreType.DMA, pltpu.SemaphoreType.DMA,
        ],
        compiler_params=pltpu.CompilerParams(use_tc_tiling_on_sc=False),
    )
    def sc_kernel(pc_hbm, nn2_hbm, out_hbm,
                  idx0, idx1, nbr0, nbr1, cen0, cen1, out0, out1,
                  sg0, sg1, so0, so1, si0, si1, sc0, sc1):
        wid = lax.axis_index("s") * num_cores + lax.axis_index("c")
        row_base = wid * rows_per_w
        idx_v, nbr_v, cen_v, out_v = (idx0, idx1), (nbr0, nbr1), (cen0, cen1), (out0, out1)
        sg, so, si, sc = (sg0, sg1), (so0, so1), (si0, si1), (sc0, sc1)

        def issue_idx(ci, bf):
            r0 = row_base + ci * rows_c
            pltpu.async_copy(nn2_hbm.at[pl.ds(r0 * K // 128, ng)], idx_v[bf], si[bf])

        def wait_idx(bf):
            pltpu.make_async_copy(nn2_hbm.at[pl.ds(0, ng)], idx_v[bf], si[bf]).wait()

        def issue_gather_cen(ci, bf):
            r0 = row_base + ci * rows_c
            for q2 in range(ng):
                pltpu.async_copy(
                    pc_hbm.at[idx_v[bf].at[q2]],
                    nbr_v[bf].at[pl.ds(q2 * 128, 128)], sg[bf])
            pltpu.async_copy(pc_hbm.at[pl.ds(r0, rows_c)], cen_v[bf], sc[bf])

        def wait_gather(bf):
            for q2 in range(ng):
                pltpu.make_async_copy(
                    pc_hbm.at[idx_v[bf].at[q2]],
                    nbr_v[bf].at[pl.ds(q2 * 128, 128)], sg[bf]).wait()

        def wait_cen(bf):
            pltpu.make_async_copy(
                pc_hbm.at[pl.ds(0, rows_c)], cen_v[bf], sc[bf]).wait()

        issue_idx(0, 0)
        issue_idx(1, 1)
        wait_idx(0)
        issue_gather_cen(0, 0)

        def outer(h, _):
            for bf in range(2):
                i = h * 2 + bf
                r0 = row_base + i * rows_c
                wait_gather(bf)
                issue_idx(jnp.minimum(i + 2, n_chunks - 1), bf)
                wait_idx(bf ^ 1)
                issue_gather_cen(jnp.minimum(i + 1, n_chunks - 1), bf ^ 1)

                @pl.when(h >= 1)
                def _():
                    pltpu.make_async_copy(
                        out_v[bf], out_hbm.at[pl.ds(r0 * K, g)], so[bf]).wait()

                wait_cen(bf)

                def edge(rr, _):
                    base = rr * K
                    cens = [cen_v[bf][rr, pl.ds(q * 16, 16)]
                            for q in range(d // 16)]
                    for kk in range(K):
                        j = base + kk
                        for q in range(d // 16):
                            nbr = nbr_v[bf][j, pl.ds(q * 16, 16)]
                            out_v[bf][j, pl.ds(q * 16, 16)] = cens[q]
                            out_v[bf][j, pl.ds(d + q * 16, 16)] = nbr - cens[q]
                    return 0

                lax.fori_loop(0, rows_c, edge, 0)
                pltpu.async_copy(out_v[bf], out_hbm.at[pl.ds(r0 * K, g)], so[bf])
            return 0

        lax.fori_loop(0, n_chunks // 2, outer, 0)
        last0 = row_base + (n_chunks - 2) * rows_c
        last1 = row_base + (n_chunks - 1) * rows_c
        pltpu.make_async_copy(out_v[0], out_hbm.at[pl.ds(last0 * K, g)], so[0]).wait()
        pltpu.make_async_copy(out_v[1], out_hbm.at[pl.ds(last1 * K, g)], so[1]).wait()
        wait_gather(0)
        wait_cen(0)
        wait_idx(1)

    return sc_kernel(pc_flat, nn2)


def kernel(inputs):
    known_axes = tuple(i for i, s in enumerate(inputs.shape) if s == 1)
    pc = jnp.squeeze(inputs, axis=known_axes) if known_axes else inputs
    b, n, d = pc.shape

    halves = []
    for h in range(2):
        pch = pc[h * (b // 2):(h + 1) * (b // 2)]
        nn_t = _topk_indices(pch, block_r=256)
        nn = jnp.transpose(nn_t, (0, 2, 1)).reshape(b // 2 * n * K)
        halves.append(_sc_gather(pch.reshape(b // 2 * n, d), nn, rows_c=16))
    out = jnp.concatenate(halves, axis=0)
    return out.reshape(b, n, K, 2 * d)
